# baseline (speedup 1.0000x reference)
import jax
import jax.numpy as jnp
from jax.experimental import pallas as pl
from jax.experimental.pallas import tpu as pltpu

_TILE = 12512


def _invres_kernel(x_ref, w1_ref, b1_ref, w2_ref, b2_ref, w3_ref, b3_ref,
                   beta_ref, out_ref, fx_ref):
    sp = jax.nn.softplus(beta_ref[0, 0])
    half_sp = 0.5 * sp
    scale = 0.5 / 1.1

    def swish(h):
        return h * (1.0 + jnp.tanh(h * half_sp)) * scale

    x = x_ref[...]
    h = jnp.dot(x, w1_ref[...], preferred_element_type=jnp.float32) + b1_ref[...]
    h = swish(h)
    h = jnp.dot(h, w2_ref[...], preferred_element_type=jnp.float32) + b2_ref[...]
    h = swish(h)
    fx = jnp.dot(h, w3_ref[...], preferred_element_type=jnp.float32) + b3_ref[...]
    fx_ref[...] = fx
    out_ref[...] = fx + x


def kernel(x, W1, b1, W2, b2, W3, b3, beta):
    B = x.shape[0]
    C = W1.shape[1]
    dim = W1.shape[0]
    n = x.shape[1] // C
    rows = B * n
    xr = x.reshape(rows, C)

    w1t = W1.T
    w2t = W2.T
    w3t = W3.T
    b1r = b1.reshape(1, dim)
    b2r = b2.reshape(1, dim)
    b3r = b3.reshape(1, C)
    betar = beta.reshape(1, 1)

    grid = (pl.cdiv(rows, _TILE),)
    full = pl.BlockSpec(lambda i: (0, 0))
    out, fx = pl.pallas_call(
        _invres_kernel,
        grid=grid,
        in_specs=[
            pl.BlockSpec((_TILE, C), lambda i: (i, 0)),
            pl.BlockSpec(w1t.shape, lambda i: (0, 0)),
            pl.BlockSpec(b1r.shape, lambda i: (0, 0)),
            pl.BlockSpec(w2t.shape, lambda i: (0, 0)),
            pl.BlockSpec(b2r.shape, lambda i: (0, 0)),
            pl.BlockSpec(w3t.shape, lambda i: (0, 0)),
            pl.BlockSpec(b3r.shape, lambda i: (0, 0)),
            pl.BlockSpec(betar.shape, lambda i: (0, 0)),
        ],
        out_specs=[
            pl.BlockSpec((_TILE, C), lambda i: (i, 0)),
            pl.BlockSpec((_TILE, C), lambda i: (i, 0)),
        ],
        out_shape=[
            jax.ShapeDtypeStruct((rows, C), jnp.float32),
            jax.ShapeDtypeStruct((rows, C), jnp.float32),
        ],
        compiler_params=pltpu.CompilerParams(
            dimension_semantics=("parallel",),
        ),
    )(xr, w1t, b1r, w2t, b2r, w3t, b3r, betar)

    return (out.reshape(B, n * C), fx.reshape(B, n, C))


# speedup vs baseline: 1.0032x; 1.0032x over previous
import jax
import jax.numpy as jnp
from jax.experimental import pallas as pl
from jax.experimental.pallas import tpu as pltpu

_TILE = 14288


def _invres_kernel(x_ref, w1_ref, b1_ref, w2_ref, b2_ref, w3_ref, b3_ref,
                   beta_ref, out_ref, fx_ref):
    sp = jax.nn.softplus(beta_ref[0, 0])
    half_sp = 0.5 * sp
    scale = 0.5 / 1.1

    def swish(h):
        return h * (1.0 + jnp.tanh(h * half_sp)) * scale

    x = x_ref[...]
    h = jnp.dot(x, w1_ref[...], preferred_element_type=jnp.float32) + b1_ref[...]
    h = swish(h)
    h = jnp.dot(h, w2_ref[...], preferred_element_type=jnp.float32) + b2_ref[...]
    h = swish(h)
    fx = jnp.dot(h, w3_ref[...], preferred_element_type=jnp.float32) + b3_ref[...]
    fx_ref[...] = fx
    out_ref[...] = fx + x


def kernel(x, W1, b1, W2, b2, W3, b3, beta):
    B = x.shape[0]
    C = W1.shape[1]
    dim = W1.shape[0]
    n = x.shape[1] // C
    rows = B * n
    xr = x.reshape(rows, C)

    w1t = W1.T
    w2t = W2.T
    w3t = W3.T
    b1r = b1.reshape(1, dim)
    b2r = b2.reshape(1, dim)
    b3r = b3.reshape(1, C)
    betar = beta.reshape(1, 1)

    grid = (pl.cdiv(rows, _TILE),)
    full = pl.BlockSpec(lambda i: (0, 0))
    out, fx = pl.pallas_call(
        _invres_kernel,
        grid=grid,
        in_specs=[
            pl.BlockSpec((_TILE, C), lambda i: (i, 0)),
            pl.BlockSpec(w1t.shape, lambda i: (0, 0)),
            pl.BlockSpec(b1r.shape, lambda i: (0, 0)),
            pl.BlockSpec(w2t.shape, lambda i: (0, 0)),
            pl.BlockSpec(b2r.shape, lambda i: (0, 0)),
            pl.BlockSpec(w3t.shape, lambda i: (0, 0)),
            pl.BlockSpec(b3r.shape, lambda i: (0, 0)),
            pl.BlockSpec(betar.shape, lambda i: (0, 0)),
        ],
        out_specs=[
            pl.BlockSpec((_TILE, C), lambda i: (i, 0)),
            pl.BlockSpec((_TILE, C), lambda i: (i, 0)),
        ],
        out_shape=[
            jax.ShapeDtypeStruct((rows, C), jnp.float32),
            jax.ShapeDtypeStruct((rows, C), jnp.float32),
        ],
        compiler_params=pltpu.CompilerParams(
            dimension_semantics=("parallel",),
        ),
    )(xr, w1t, b1r, w2t, b2r, w3t, b3r, betar)

    return (out.reshape(B, n * C), fx.reshape(B, n, C))
